# baseline (speedup 1.0000x reference)
import jax
import jax.numpy as jnp
from jax.experimental import pallas as pl
from jax.experimental.pallas import tpu as pltpu

ALPHA = 1.702
LIMIT = 7.0

_TT = 1024


def _body(x_ref, w1_ref, b1_ref, wd_ref, bd_ref, rw_ref, o_ref, acc_ref):
    e = pl.program_id(0)
    t = pl.program_id(1)
    ne = pl.num_programs(0)
    d = wd_ref.shape[1]
    sl = pl.ds(t * _TT, _TT)
    x = x_ref[sl, :].astype(jnp.bfloat16)
    gu = jnp.dot(x, w1_ref[0], preferred_element_type=jnp.float32) + b1_ref[0]
    gate = jnp.minimum(gu[:, :d], LIMIT)
    up = jnp.clip(gu[:, d:], -LIMIT, LIMIT)
    glu = gate * jax.nn.sigmoid(gate * ALPHA)
    gated = ((up + 1.0) * glu).astype(jnp.bfloat16)
    out = jnp.dot(gated, wd_ref[0], preferred_element_type=jnp.float32) + bd_ref[0]
    contrib = out * rw_ref[0, sl, :]

    @pl.when(e == 0)
    def _():
        acc_ref[sl, :] = contrib

    @pl.when((e > 0) & (e < ne - 1))
    def _():
        acc_ref[sl, :] += contrib

    @pl.when(e == ne - 1)
    def _():
        o_ref[...] = acc_ref[sl, :] + contrib


def kernel(hidden_states, router_indices, routing_weights, gate_up_proj,
           gate_up_proj_bias, down_proj, down_proj_bias):
    del router_indices
    T, H = hidden_states.shape
    E, _, D2 = gate_up_proj.shape
    D = D2 // 2

    w1 = jnp.transpose(gate_up_proj.astype(jnp.bfloat16).reshape(E, H, D, 2),
                       (0, 1, 3, 2)).reshape(E, H, D2)
    b1 = jnp.transpose(gate_up_proj_bias.reshape(E, D, 2),
                       (0, 2, 1)).reshape(E, 1, D2)
    wd = down_proj.astype(jnp.bfloat16)
    bd = down_proj_bias[:, None, :]
    rw = jnp.transpose(routing_weights)[:, :, None]

    num_t = T // _TT
    grid = (E, num_t)

    def out_idx(e, t):
        return (jnp.where(e == E - 1, t, 0), 0)

    return pl.pallas_call(
        _body,
        grid=grid,
        in_specs=[
            pl.BlockSpec((T, H), lambda e, t: (0, 0)),
            pl.BlockSpec((1, H, D2), lambda e, t: (e, 0, 0)),
            pl.BlockSpec((1, 1, D2), lambda e, t: (e, 0, 0)),
            pl.BlockSpec((1, D, H), lambda e, t: (e, 0, 0)),
            pl.BlockSpec((1, 1, H), lambda e, t: (e, 0, 0)),
            pl.BlockSpec((1, T, 1), lambda e, t: (e, 0, 0)),
        ],
        out_specs=pl.BlockSpec((_TT, H), out_idx),
        out_shape=jax.ShapeDtypeStruct((T, H), jnp.float32),
        scratch_shapes=[pltpu.VMEM((T, H), jnp.float32)],
        compiler_params=pltpu.CompilerParams(
            dimension_semantics=("arbitrary", "arbitrary"),
        ),
    )(hidden_states, w1, b1, wd, bd, rw)


# speedup vs baseline: 1.1731x; 1.1731x over previous
import jax
import jax.numpy as jnp
from jax.experimental import pallas as pl
from jax.experimental.pallas import tpu as pltpu

ALPHA = 1.702
LIMIT = 7.0

_TT = 1024


def _body(x_ref, wg_ref, wu_ref, bg_ref, bu_ref, wd_ref, bd_ref, rw_ref,
          o_ref, acc_ref):
    e = pl.program_id(0)
    t = pl.program_id(1)
    ne = pl.num_programs(0)
    sl = pl.ds(t * _TT, _TT)
    x = x_ref[sl, :].astype(jnp.bfloat16)
    gate = jnp.dot(x, wg_ref[0], preferred_element_type=jnp.float32) + bg_ref[0]
    up = jnp.dot(x, wu_ref[0], preferred_element_type=jnp.float32) + bu_ref[0]
    gate = jnp.minimum(gate, LIMIT)
    up = jnp.clip(up, -LIMIT, LIMIT)
    glu = gate * jax.nn.sigmoid(gate * ALPHA)
    gated = ((up + 1.0) * glu).astype(jnp.bfloat16)
    out = jnp.dot(gated, wd_ref[0], preferred_element_type=jnp.float32) + bd_ref[0]
    contrib = out * rw_ref[0, sl, :]

    @pl.when(e == 0)
    def _():
        acc_ref[sl, :] = contrib

    @pl.when((e > 0) & (e < ne - 1))
    def _():
        acc_ref[sl, :] += contrib

    @pl.when(e == ne - 1)
    def _():
        o_ref[...] = acc_ref[sl, :] + contrib


def kernel(hidden_states, router_indices, routing_weights, gate_up_proj,
           gate_up_proj_bias, down_proj, down_proj_bias):
    del router_indices
    T, H = hidden_states.shape
    E, _, D2 = gate_up_proj.shape
    D = D2 // 2

    wgu = jnp.transpose(gate_up_proj.astype(jnp.bfloat16).reshape(E, H, D, 2),
                        (3, 0, 1, 2))
    wg = wgu[0]
    wu = wgu[1]
    wd = down_proj.astype(jnp.bfloat16)
    bg = gate_up_proj_bias[:, None, 0::2]
    bu = gate_up_proj_bias[:, None, 1::2]
    bd = down_proj_bias[:, None, :]
    rw = jnp.transpose(routing_weights)[:, :, None]

    num_t = T // _TT
    grid = (E, num_t)
    last_t = num_t - 1

    def out_idx(e, t):
        return (jnp.where(e == E - 1, t, 0), 0)

    return pl.pallas_call(
        _body,
        grid=grid,
        in_specs=[
            pl.BlockSpec((T, H), lambda e, t: (0, 0)),
            pl.BlockSpec((1, H, D), lambda e, t: (e, 0, 0)),
            pl.BlockSpec((1, H, D), lambda e, t: (e, 0, 0)),
            pl.BlockSpec((1, 1, D), lambda e, t: (e, 0, 0)),
            pl.BlockSpec((1, 1, D), lambda e, t: (e, 0, 0)),
            pl.BlockSpec((1, D, H), lambda e, t: (e, 0, 0)),
            pl.BlockSpec((1, 1, H), lambda e, t: (e, 0, 0)),
            pl.BlockSpec((1, T, 1), lambda e, t: (e, 0, 0)),
        ],
        out_specs=pl.BlockSpec((_TT, H), out_idx),
        out_shape=jax.ShapeDtypeStruct((T, H), jnp.float32),
        scratch_shapes=[pltpu.VMEM((T, H), jnp.float32)],
        compiler_params=pltpu.CompilerParams(
            dimension_semantics=("arbitrary", "arbitrary"),
        ),
    )(hidden_states, wg, wu, bg, bu, wd, bd, rw)
